# baseline (speedup 1.0000x reference)
import jax
import jax.numpy as jnp
import numpy as np
from jax import lax
from jax.experimental import pallas as pl
from jax.experimental.pallas import tpu as pltpu
from jax.experimental.pallas import tpu_sc as plsc

N = 128
K = N * (N + 1) // 2
NROWS = 1024 * 8
NC, NS, L = 2, 16, 16
NW = NC * NS
ROWS_PER_W = NROWS // NW
NCHUNK = K // L

_r, _c = np.triu_indices(N)
_TRIU_IDX = (_r.astype(np.int32) * N + _c.astype(np.int32))


def _sc_body(x_hbm, idx_hbm, o_hbm, idx_v, xbuf, obuf):
    wid = lax.axis_index("s") * NC + lax.axis_index("c")
    base = wid * ROWS_PER_W
    pltpu.sync_copy(idx_hbm, idx_v)

    def per_row(m, carry):
        r = base + m
        pltpu.sync_copy(x_hbm.at[r], xbuf)

        def chunk(k, carry2):
            ids = idx_v[pl.ds(k * L, L)]
            obuf[pl.ds(k * L, L)] = plsc.load_gather(xbuf, [ids])
            return carry2

        lax.fori_loop(0, NCHUNK, chunk, 0, unroll=4)
        pltpu.sync_copy(obuf, o_hbm.at[r])
        return carry

    lax.fori_loop(0, ROWS_PER_W, per_row, 0)


@jax.jit
def _spd_vectorize(x_flat, idx):
    mesh = plsc.VectorSubcoreMesh(core_axis_name="c", subcore_axis_name="s")
    return pl.kernel(
        _sc_body,
        out_type=jax.ShapeDtypeStruct((NROWS, K), jnp.float32),
        mesh=mesh,
        compiler_params=pltpu.CompilerParams(needs_layout_passes=False),
        scratch_types=[
            pltpu.VMEM((K,), jnp.int32),
            pltpu.VMEM((N * N,), jnp.float32),
            pltpu.VMEM((K,), jnp.float32),
        ],
    )(x_flat, idx)


def kernel(input):
    x_flat = input.reshape(NROWS, N * N)
    idx = jnp.asarray(_TRIU_IDX)
    out = _spd_vectorize(x_flat, idx)
    return out.reshape(1024, 8, K)


# speedup vs baseline: 2.3836x; 2.3836x over previous
import jax
import jax.numpy as jnp
import numpy as np
from jax import lax
from jax.experimental import pallas as pl
from jax.experimental.pallas import tpu as pltpu
from jax.experimental.pallas import tpu_sc as plsc

N = 128
K = N * (N + 1) // 2
NROWS = 1024 * 8
NC, NS, L = 2, 16, 16
NW = NC * NS
ROWS_PER_W = NROWS // NW
NPAIR = ROWS_PER_W // 2
NCHUNK = K // L
UNROLL = 12

_r, _c = np.triu_indices(N)
_TRIU_IDX = (_r.astype(np.int32) * N + _c.astype(np.int32))


def _sc_body(x_hbm, idx_hbm, o_hbm, idx_v, xbuf0, xbuf1, obuf0, obuf1,
             gsem0, gsem1, ssem0, ssem1):
    wid = lax.axis_index("s") * NC + lax.axis_index("c")
    base = wid * ROWS_PER_W
    pltpu.sync_copy(idx_hbm, idx_v)

    def pack_row(xbuf, obuf):
        @plsc.parallel_loop(0, NCHUNK, 1, unroll=UNROLL)
        def _(k):
            ids = idx_v[pl.ds(k * L, L)]
            obuf[pl.ds(k * L, L)] = plsc.load_gather(xbuf, [ids])

    slots = ((xbuf0, obuf0, gsem0, ssem0), (xbuf1, obuf1, gsem1, ssem1))

    pltpu.async_copy(x_hbm.at[base], xbuf0, gsem0)
    pltpu.async_copy(x_hbm.at[base + 1], xbuf1, gsem1)

    def pair(p, carry):
        m0 = base + 2 * p
        for b, (xbuf, obuf, gsem_b, ssem_b) in enumerate(slots):
            m = m0 + b
            pltpu.make_async_copy(x_hbm.at[m], xbuf, gsem_b).wait()

            @pl.when(p > 0)
            def _():
                pltpu.make_async_copy(obuf, o_hbm.at[m - 2], ssem_b).wait()

            pack_row(xbuf, obuf)
            pltpu.async_copy(obuf, o_hbm.at[m], ssem_b)

            @pl.when(p + 1 < NPAIR)
            def _():
                pltpu.async_copy(x_hbm.at[m + 2], xbuf, gsem_b)
        return carry

    lax.fori_loop(0, NPAIR, pair, 0)
    last = base + ROWS_PER_W
    pltpu.make_async_copy(obuf0, o_hbm.at[last - 2], ssem0).wait()
    pltpu.make_async_copy(obuf1, o_hbm.at[last - 1], ssem1).wait()


@jax.jit
def _spd_vectorize(x_flat, idx):
    mesh = plsc.VectorSubcoreMesh(core_axis_name="c", subcore_axis_name="s")
    return pl.kernel(
        _sc_body,
        out_type=jax.ShapeDtypeStruct((NROWS, K), jnp.float32),
        mesh=mesh,
        compiler_params=pltpu.CompilerParams(needs_layout_passes=False),
        scratch_types=[
            pltpu.VMEM((K,), jnp.int32),
            pltpu.VMEM((N * N,), jnp.float32),
            pltpu.VMEM((N * N,), jnp.float32),
            pltpu.VMEM((K,), jnp.float32),
            pltpu.VMEM((K,), jnp.float32),
            pltpu.SemaphoreType.DMA,
            pltpu.SemaphoreType.DMA,
            pltpu.SemaphoreType.DMA,
            pltpu.SemaphoreType.DMA,
        ],
    )(x_flat, idx)


def kernel(input):
    x_flat = input.reshape(NROWS, N * N)
    idx = jnp.asarray(_TRIU_IDX)
    out = _spd_vectorize(x_flat, idx)
    return out.reshape(1024, 8, K)


# speedup vs baseline: 3.7505x; 1.5734x over previous
import jax
import jax.numpy as jnp
import numpy as np
from jax import lax
from jax.experimental import pallas as pl
from jax.experimental.pallas import tpu as pltpu
from jax.experimental.pallas import tpu_sc as plsc

N = 128
K = N * (N + 1) // 2
B, C = 1024, 8
NROWS = B * C
NC, NS, L = 2, 16, 16
NW = NC * NS
ROWS_PER_W = NROWS // NW
NPAIR = ROWS_PER_W // 2
NCHUNK = K // L
UNROLL = 12

_r, _c = np.triu_indices(N)
_TRIU_IDX = (_r.astype(np.int32) * N + _c.astype(np.int32))


def _sc_body(x_hbm, idx_hbm, o_hbm, idx_v, xbuf0, xbuf1, obuf0, obuf1,
             gsem0, gsem1, ssem0, ssem1):
    wid = lax.axis_index("s") * NC + lax.axis_index("c")
    base = wid * ROWS_PER_W
    pltpu.sync_copy(idx_hbm, idx_v)

    def pack_row(xbuf, obuf):
        @plsc.parallel_loop(0, NCHUNK, 1, unroll=UNROLL)
        def _(k):
            ids = idx_v[pl.ds(k * L, L)]
            vals = plsc.load_gather(xbuf, [ids >> 7, ids & 127])
            obuf[pl.ds(k * L, L)] = vals

    slots = ((xbuf0, obuf0, gsem0, ssem0), (xbuf1, obuf1, gsem1, ssem1))

    def src(m):
        return x_hbm.at[m // C, m % C]

    def dst(m):
        return o_hbm.at[m // C, m % C]

    pltpu.async_copy(src(base), xbuf0, gsem0)
    pltpu.async_copy(src(base + 1), xbuf1, gsem1)

    def pair(p, carry):
        m0 = base + 2 * p
        for b, (xbuf, obuf, gsem_b, ssem_b) in enumerate(slots):
            m = m0 + b
            pltpu.make_async_copy(src(m), xbuf, gsem_b).wait()

            @pl.when(p > 0)
            def _():
                pltpu.make_async_copy(obuf, dst(m - 2), ssem_b).wait()

            pack_row(xbuf, obuf)
            pltpu.async_copy(obuf, dst(m), ssem_b)

            @pl.when(p + 1 < NPAIR)
            def _():
                pltpu.async_copy(src(m + 2), xbuf, gsem_b)
        return carry

    lax.fori_loop(0, NPAIR, pair, 0)
    last = base + ROWS_PER_W
    pltpu.make_async_copy(obuf0, dst(last - 2), ssem0).wait()
    pltpu.make_async_copy(obuf1, dst(last - 1), ssem1).wait()


@jax.jit
def _spd_vectorize(x, idx):
    mesh = plsc.VectorSubcoreMesh(core_axis_name="c", subcore_axis_name="s")
    return pl.kernel(
        _sc_body,
        out_type=jax.ShapeDtypeStruct((B, C, K), jnp.float32),
        mesh=mesh,
        compiler_params=pltpu.CompilerParams(needs_layout_passes=False),
        scratch_types=[
            pltpu.VMEM((K,), jnp.int32),
            pltpu.VMEM((N, N), jnp.float32),
            pltpu.VMEM((N, N), jnp.float32),
            pltpu.VMEM((K,), jnp.float32),
            pltpu.VMEM((K,), jnp.float32),
            pltpu.SemaphoreType.DMA,
            pltpu.SemaphoreType.DMA,
            pltpu.SemaphoreType.DMA,
            pltpu.SemaphoreType.DMA,
        ],
    )(x, idx)


def kernel(input):
    idx = jnp.asarray(_TRIU_IDX)
    return _spd_vectorize(input, idx)


# speedup vs baseline: 3.8638x; 1.0302x over previous
import jax
import jax.numpy as jnp
from jax import lax
from jax.experimental import pallas as pl
from jax.experimental.pallas import tpu as pltpu
from jax.experimental.pallas import tpu_sc as plsc

N = 128
K = N * (N + 1) // 2
B, C = 1024, 8
NROWS = B * C
NC, NS, L = 2, 16, 16
NW = NC * NS
ROWS_PER_W = NROWS // NW
NPAIR = ROWS_PER_W // 2

_OFF = [r * N - r * (r - 1) // 2 for r in range(N)]


def _sc_body(x_hbm, o_hbm, xbuf0, xbuf1, obuf0, obuf1, gsem0, gsem1,
             ssem0, ssem1):
    wid = lax.axis_index("s") * NC + lax.axis_index("c")
    base = wid * ROWS_PER_W

    lanes = lax.iota(jnp.int32, L)

    def pack_row(xbuf, obuf):
        for r in range(113):
            w = N - r
            nj = (w + L - 1) // L
            for j in range(nj):
                cs = min(r + L * j, N - L)
                obuf[pl.ds(_OFF[r] + cs - r, L)] = xbuf[r, pl.ds(cs, L)]
        for r in range(113, N):
            vals = xbuf[r, pl.ds(N - L, L)]
            plsc.store_scatter(
                obuf,
                [lanes + (_OFF[r] + N - L - r)],
                vals,
                mask=lanes >= (r - (N - L)),
            )

    slots = ((xbuf0, obuf0, gsem0, ssem0), (xbuf1, obuf1, gsem1, ssem1))

    def src(m):
        return x_hbm.at[m // C, m % C]

    def dst(m):
        return o_hbm.at[m // C, m % C]

    pltpu.async_copy(src(base), xbuf0, gsem0)
    pltpu.async_copy(src(base + 1), xbuf1, gsem1)

    def pair(p, carry):
        m0 = base + 2 * p
        for b, (xbuf, obuf, gsem_b, ssem_b) in enumerate(slots):
            m = m0 + b
            pltpu.make_async_copy(src(m), xbuf, gsem_b).wait()

            @pl.when(p > 0)
            def _():
                pltpu.make_async_copy(obuf, dst(m - 2), ssem_b).wait()

            pack_row(xbuf, obuf)
            pltpu.async_copy(obuf, dst(m), ssem_b)

            @pl.when(p + 1 < NPAIR)
            def _():
                pltpu.async_copy(src(m + 2), xbuf, gsem_b)
        return carry

    lax.fori_loop(0, NPAIR, pair, 0)
    last = base + ROWS_PER_W
    pltpu.make_async_copy(obuf0, dst(last - 2), ssem0).wait()
    pltpu.make_async_copy(obuf1, dst(last - 1), ssem1).wait()


@jax.jit
def _spd_vectorize(x):
    mesh = plsc.VectorSubcoreMesh(core_axis_name="c", subcore_axis_name="s")
    return pl.kernel(
        _sc_body,
        out_type=jax.ShapeDtypeStruct((B, C, K), jnp.float32),
        mesh=mesh,
        compiler_params=pltpu.CompilerParams(needs_layout_passes=False),
        scratch_types=[
            pltpu.VMEM((N, N), jnp.float32),
            pltpu.VMEM((N, N), jnp.float32),
            pltpu.VMEM((K,), jnp.float32),
            pltpu.VMEM((K,), jnp.float32),
            pltpu.SemaphoreType.DMA,
            pltpu.SemaphoreType.DMA,
            pltpu.SemaphoreType.DMA,
            pltpu.SemaphoreType.DMA,
        ],
    )(x)


def kernel(input):
    return _spd_vectorize(input)


# speedup vs baseline: 3.9209x; 1.0148x over previous
import jax
import jax.numpy as jnp
from jax import lax
from jax.experimental import pallas as pl
from jax.experimental.pallas import tpu as pltpu
from jax.experimental.pallas import tpu_sc as plsc

N = 128
K = N * (N + 1) // 2
B, C = 1024, 8
NROWS = B * C
NC, NS, L = 2, 16, 16
NW = NC * NS
ROWS_PER_W = NROWS // NW
NPAIR = ROWS_PER_W // 2

_OFF = [r * N - r * (r - 1) // 2 for r in range(N)]


def _sc_body(x_hbm, o_hbm, xbuf0, xbuf1, xbuf2, xbuf3, obuf0, obuf1,
             obuf2, obuf3, gsem0, gsem1, gsem2, gsem3, ssem0, ssem1,
             ssem2, ssem3):
    wid = lax.axis_index("s") * NC + lax.axis_index("c")
    base = wid * ROWS_PER_W

    lanes = lax.iota(jnp.int32, L)

    def pack_row(xbuf, obuf):
        for r in range(113):
            w = N - r
            nj = (w + L - 1) // L
            for j in range(nj):
                cs = min(r + L * j, N - L)
                obuf[pl.ds(_OFF[r] + cs - r, L)] = xbuf[r, pl.ds(cs, L)]
        for r in range(113, N):
            vals = xbuf[r, pl.ds(N - L, L)]
            plsc.store_scatter(
                obuf,
                [lanes + (_OFF[r] + N - L - r)],
                vals,
                mask=lanes >= (r - (N - L)),
            )

    slots = (
        (xbuf0, obuf0, gsem0, ssem0),
        (xbuf1, obuf1, gsem1, ssem1),
        (xbuf2, obuf2, gsem2, ssem2),
        (xbuf3, obuf3, gsem3, ssem3),
    )
    nslot = len(slots)
    nstep = ROWS_PER_W // nslot

    def src(m):
        return x_hbm.at[m // C, m % C]

    def dst(m):
        return o_hbm.at[m // C, m % C]

    for b, (xbuf, _, gsem_b, _) in enumerate(slots):
        pltpu.async_copy(src(base + b), xbuf, gsem_b)

    def step(p, carry):
        m0 = base + nslot * p
        for b, (xbuf, obuf, gsem_b, ssem_b) in enumerate(slots):
            m = m0 + b
            pltpu.make_async_copy(src(m), xbuf, gsem_b).wait()

            @pl.when(p > 0)
            def _():
                pltpu.make_async_copy(obuf, dst(m - nslot), ssem_b).wait()

            pack_row(xbuf, obuf)
            pltpu.async_copy(obuf, dst(m), ssem_b)

            @pl.when(p + 1 < nstep)
            def _():
                pltpu.async_copy(src(m + nslot), xbuf, gsem_b)
        return carry

    lax.fori_loop(0, nstep, step, 0)
    last = base + ROWS_PER_W
    for b, (_, obuf, _, ssem_b) in enumerate(slots):
        pltpu.make_async_copy(obuf, dst(last - nslot + b), ssem_b).wait()


@jax.jit
def _spd_vectorize(x):
    mesh = plsc.VectorSubcoreMesh(core_axis_name="c", subcore_axis_name="s")
    return pl.kernel(
        _sc_body,
        out_type=jax.ShapeDtypeStruct((B, C, K), jnp.float32),
        mesh=mesh,
        compiler_params=pltpu.CompilerParams(needs_layout_passes=False),
        scratch_types=(
            [pltpu.VMEM((N, N), jnp.float32)] * 4
            + [pltpu.VMEM((K,), jnp.float32)] * 4
            + [pltpu.SemaphoreType.DMA] * 8
        ),
    )(x)


def kernel(input):
    return _spd_vectorize(input)


# speedup vs baseline: 4.2071x; 1.0730x over previous
import jax
import jax.numpy as jnp
from jax import lax
from jax.experimental import pallas as pl
from jax.experimental.pallas import tpu as pltpu
from jax.experimental.pallas import tpu_sc as plsc

N = 128
K = N * (N + 1) // 2
B, C = 1024, 8
NC, NS, L = 2, 16, 16
NW = NC * NS
CPC = 2
NCHUNKS = C // CPC
MATS = B * CPC
MATS_PER_W = MATS // NW
NSLOT = 4
NSTEP = MATS_PER_W // NSLOT

_OFF = [r * N - r * (r - 1) // 2 for r in range(N)]


def _make_sc_body(c0):
    def _sc_body(x_hbm, o_hbm, xbuf0, xbuf1, xbuf2, xbuf3, obuf0, obuf1,
                 obuf2, obuf3, gsem0, gsem1, gsem2, gsem3, ssem0, ssem1,
                 ssem2, ssem3):
        wid = lax.axis_index("s") * NC + lax.axis_index("c")
        base = wid * MATS_PER_W

        lanes = lax.iota(jnp.int32, L)

        def pack_row(xbuf, obuf):
            for r in range(113):
                w = N - r
                nj = (w + L - 1) // L
                for j in range(nj):
                    cs = min(r + L * j, N - L)
                    obuf[pl.ds(_OFF[r] + cs - r, L)] = xbuf[r, pl.ds(cs, L)]
            for r in range(113, N):
                vals = xbuf[r, pl.ds(N - L, L)]
                plsc.store_scatter(
                    obuf,
                    [lanes + (_OFF[r] + N - L - r)],
                    vals,
                    mask=lanes >= (r - (N - L)),
                )

        slots = (
            (xbuf0, obuf0, gsem0, ssem0),
            (xbuf1, obuf1, gsem1, ssem1),
            (xbuf2, obuf2, gsem2, ssem2),
            (xbuf3, obuf3, gsem3, ssem3),
        )

        def src(m):
            return x_hbm.at[m // CPC, c0 + m % CPC]

        def dst(m):
            return o_hbm.at[m // CPC, m % CPC]

        for b, (xbuf, _, gsem_b, _) in enumerate(slots):
            pltpu.async_copy(src(base + b), xbuf, gsem_b)

        def step(p, carry):
            m0 = base + NSLOT * p
            for b, (xbuf, obuf, gsem_b, ssem_b) in enumerate(slots):
                m = m0 + b
                pltpu.make_async_copy(src(m), xbuf, gsem_b).wait()

                @pl.when(p > 0)
                def _():
                    pltpu.make_async_copy(obuf, dst(m - NSLOT), ssem_b).wait()

                pack_row(xbuf, obuf)
                pltpu.async_copy(obuf, dst(m), ssem_b)

                @pl.when(p + 1 < NSTEP)
                def _():
                    pltpu.async_copy(src(m + NSLOT), xbuf, gsem_b)
            return carry

        lax.fori_loop(0, NSTEP, step, 0)
        last = base + MATS_PER_W
        for b, (_, obuf, _, ssem_b) in enumerate(slots):
            pltpu.make_async_copy(obuf, dst(last - NSLOT + b), ssem_b).wait()

    return _sc_body


def _sc_chunk(x, c0):
    mesh = plsc.VectorSubcoreMesh(core_axis_name="c", subcore_axis_name="s")
    return pl.kernel(
        _make_sc_body(c0),
        out_type=jax.ShapeDtypeStruct((B, CPC, K), jnp.float32),
        mesh=mesh,
        compiler_params=pltpu.CompilerParams(needs_layout_passes=False),
        scratch_types=(
            [pltpu.VMEM((N, N), jnp.float32)] * 4
            + [pltpu.VMEM((K,), jnp.float32)] * 4
            + [pltpu.SemaphoreType.DMA] * 8
        ),
        name=f"spd_pack_c{c0}",
    )(x)


BB = 512
KB = 512


def _tc_t_first_body(chunk_ref, out_ref):
    for cc in range(CPC):
        out_ref[cc] = jnp.transpose(chunk_ref[:, cc, :])


def _tc_t_body(buf_ref, chunk_ref, out_ref):
    del buf_ref
    for cc in range(CPC):
        out_ref[cc] = jnp.transpose(chunk_ref[:, cc, :])


def _tc_transpose(chunk, c0, buf):
    grid = ((K + KB - 1) // KB, B // BB)
    in_spec = pl.BlockSpec((BB, CPC, KB), lambda k, b: (b, 0, k))
    out_spec = pl.BlockSpec((CPC, KB, BB), lambda k, b: (c0 // CPC, k, b))
    if buf is None:
        return pl.pallas_call(
            _tc_t_first_body,
            grid=grid,
            in_specs=[in_spec],
            out_specs=out_spec,
            out_shape=jax.ShapeDtypeStruct((C, K, B), jnp.float32),
        )(chunk)
    return pl.pallas_call(
        _tc_t_body,
        grid=grid,
        in_specs=[pl.BlockSpec(memory_space=pltpu.MemorySpace.HBM), in_spec],
        out_specs=out_spec,
        out_shape=jax.ShapeDtypeStruct((C, K, B), jnp.float32),
        input_output_aliases={0: 0},
    )(buf, chunk)


@jax.jit
def _spd_vectorize(x):
    buf = None
    for i in range(NCHUNKS):
        chunk = _sc_chunk(x, i * CPC)
        buf = _tc_transpose(chunk, i * CPC, buf)
    return jnp.transpose(buf, (2, 0, 1))


def kernel(input):
    return _spd_vectorize(input)


# speedup vs baseline: 4.3036x; 1.0229x over previous
import jax
import jax.numpy as jnp
from jax import lax
from jax.experimental import pallas as pl
from jax.experimental.pallas import tpu as pltpu
from jax.experimental.pallas import tpu_sc as plsc

N = 128
K = N * (N + 1) // 2
B, C = 1024, 8
NC, NS, L = 2, 16, 16
NW = NC * NS
CPC = 2
NCHUNKS = C // CPC
MATS = B * CPC
MATS_PER_W = MATS // NW
NSLOT = 4
NSTEP = MATS_PER_W // NSLOT

_OFF = [r * N - r * (r - 1) // 2 for r in range(N)]


def _make_sc_body(c0):
    def _sc_body(x_hbm, o_hbm, xbuf0, xbuf1, xbuf2, xbuf3, obuf0, obuf1,
                 obuf2, obuf3, gsem0, gsem1, gsem2, gsem3, ssem0, ssem1,
                 ssem2, ssem3):
        wid = lax.axis_index("s") * NC + lax.axis_index("c")
        base = wid * MATS_PER_W

        lanes = lax.iota(jnp.int32, L)

        def pack_row(xbuf, obuf):
            for r in range(113):
                w = N - r
                nj = (w + L - 1) // L
                for j in range(nj):
                    cs = min(r + L * j, N - L)
                    obuf[pl.ds(_OFF[r] + cs - r, L)] = xbuf[r, pl.ds(cs, L)]
            for r in range(113, N):
                vals = xbuf[r, pl.ds(N - L, L)]
                plsc.store_scatter(
                    obuf,
                    [lanes + (_OFF[r] + N - L - r)],
                    vals,
                    mask=lanes >= (r - (N - L)),
                )

        slots = (
            (xbuf0, obuf0, gsem0, ssem0),
            (xbuf1, obuf1, gsem1, ssem1),
            (xbuf2, obuf2, gsem2, ssem2),
            (xbuf3, obuf3, gsem3, ssem3),
        )

        def src(m):
            return x_hbm.at[m // CPC, c0 + m % CPC]

        def dst(m):
            return o_hbm.at[m // CPC, m % CPC]

        for b, (xbuf, _, gsem_b, _) in enumerate(slots):
            pltpu.async_copy(src(base + b), xbuf, gsem_b)

        def step(p, carry):
            m0 = base + NSLOT * p
            for b, (xbuf, obuf, gsem_b, ssem_b) in enumerate(slots):
                m = m0 + b
                pltpu.make_async_copy(src(m), xbuf, gsem_b).wait()

                @pl.when(p > 0)
                def _():
                    pltpu.make_async_copy(obuf, dst(m - NSLOT), ssem_b).wait()

                pack_row(xbuf, obuf)
                pltpu.async_copy(obuf, dst(m), ssem_b)

                @pl.when(p + 1 < NSTEP)
                def _():
                    pltpu.async_copy(src(m + NSLOT), xbuf, gsem_b)
            return carry

        lax.fori_loop(0, NSTEP, step, 0)
        last = base + MATS_PER_W
        for b, (_, obuf, _, ssem_b) in enumerate(slots):
            pltpu.make_async_copy(obuf, dst(last - NSLOT + b), ssem_b).wait()

    return _sc_body


def _sc_chunk(x, c0):
    mesh = plsc.VectorSubcoreMesh(core_axis_name="c", subcore_axis_name="s")
    return pl.kernel(
        _make_sc_body(c0),
        out_type=jax.ShapeDtypeStruct((B, CPC, K), jnp.float32),
        mesh=mesh,
        compiler_params=pltpu.CompilerParams(needs_layout_passes=False),
        scratch_types=(
            [pltpu.VMEM((N, N), jnp.float32)] * 4
            + [pltpu.VMEM((K,), jnp.float32)] * 4
            + [pltpu.SemaphoreType.DMA] * 8
        ),
        name=f"spd_pack_c{c0}",
    )(x)


BB = 512
KB = 1024


def _tc_t_first_body(chunk_ref, out_ref):
    for cc in range(CPC):
        out_ref[cc] = jnp.transpose(chunk_ref[:, cc, :])


def _tc_t_body(buf_ref, chunk_ref, out_ref):
    del buf_ref
    for cc in range(CPC):
        out_ref[cc] = jnp.transpose(chunk_ref[:, cc, :])


def _tc_transpose(chunk, c0, buf):
    grid = ((K + KB - 1) // KB, B // BB)
    in_spec = pl.BlockSpec((BB, CPC, KB), lambda k, b: (b, 0, k))
    out_spec = pl.BlockSpec((CPC, KB, BB), lambda k, b: (c0 // CPC, k, b))
    if buf is None:
        return pl.pallas_call(
            _tc_t_first_body,
            grid=grid,
            in_specs=[in_spec],
            out_specs=out_spec,
            out_shape=jax.ShapeDtypeStruct((C, K, B), jnp.float32),
        )(chunk)
    return pl.pallas_call(
        _tc_t_body,
        grid=grid,
        in_specs=[pl.BlockSpec(memory_space=pltpu.MemorySpace.HBM), in_spec],
        out_specs=out_spec,
        out_shape=jax.ShapeDtypeStruct((C, K, B), jnp.float32),
        input_output_aliases={0: 0},
    )(buf, chunk)


@jax.jit
def _spd_vectorize(x):
    buf = None
    for i in range(NCHUNKS):
        chunk = _sc_chunk(x, i * CPC)
        buf = _tc_transpose(chunk, i * CPC, buf)
    return jnp.transpose(buf, (2, 0, 1))


def kernel(input):
    return _spd_vectorize(input)


# speedup vs baseline: 4.3325x; 1.0067x over previous
import jax
import jax.numpy as jnp
from jax import lax
from jax.experimental import pallas as pl
from jax.experimental.pallas import tpu as pltpu
from jax.experimental.pallas import tpu_sc as plsc

N = 128
K = N * (N + 1) // 2
B, C = 1024, 8
NC, NS, L = 2, 16, 16
NW = NC * NS
CPC = 2
NCHUNKS = C // CPC
MATS = B * CPC
MATS_PER_W = MATS // NW
NSLOT = 4
NSTEP = MATS_PER_W // NSLOT

_OFF = [r * N - r * (r - 1) // 2 for r in range(N)]


def _make_sc_body(c0):
    def _sc_body(x_hbm, o_hbm, xbuf0, xbuf1, xbuf2, xbuf3, obuf0, obuf1,
                 obuf2, obuf3, gsem0, gsem1, gsem2, gsem3, ssem0, ssem1,
                 ssem2, ssem3):
        wid = lax.axis_index("s") * NC + lax.axis_index("c")
        base = wid * MATS_PER_W

        lanes = lax.iota(jnp.int32, L)

        def pack_row(xbuf, obuf):
            for r in range(113):
                w = N - r
                nj = (w + L - 1) // L
                for j in range(nj):
                    cs = min(r + L * j, N - L)
                    obuf[pl.ds(_OFF[r] + cs - r, L)] = xbuf[r, pl.ds(cs, L)]
            for r in range(113, N):
                vals = xbuf[r, pl.ds(N - L, L)]
                plsc.store_scatter(
                    obuf,
                    [lanes + (_OFF[r] + N - L - r)],
                    vals,
                    mask=lanes >= (r - (N - L)),
                )

        slots = (
            (xbuf0, obuf0, gsem0, ssem0),
            (xbuf1, obuf1, gsem1, ssem1),
            (xbuf2, obuf2, gsem2, ssem2),
            (xbuf3, obuf3, gsem3, ssem3),
        )

        def src(m):
            return x_hbm.at[m // CPC, c0 + m % CPC]

        def dst(m):
            return o_hbm.at[m // CPC, m % CPC]

        for b, (xbuf, _, gsem_b, _) in enumerate(slots):
            pltpu.async_copy(src(base + b), xbuf, gsem_b)

        def step(p, carry):
            m0 = base + NSLOT * p
            for b, (xbuf, obuf, gsem_b, ssem_b) in enumerate(slots):
                m = m0 + b
                pltpu.make_async_copy(src(m), xbuf, gsem_b).wait()

                @pl.when(p > 0)
                def _():
                    pltpu.make_async_copy(obuf, dst(m - NSLOT), ssem_b).wait()

                pack_row(xbuf, obuf)
                pltpu.async_copy(obuf, dst(m), ssem_b)

                @pl.when(p + 1 < NSTEP)
                def _():
                    pltpu.async_copy(src(m + NSLOT), xbuf, gsem_b)
            return carry

        lax.fori_loop(0, NSTEP, step, 0)
        last = base + MATS_PER_W
        for b, (_, obuf, _, ssem_b) in enumerate(slots):
            pltpu.make_async_copy(obuf, dst(last - NSLOT + b), ssem_b).wait()

    return _sc_body


def _sc_chunk(x, c0):
    mesh = plsc.VectorSubcoreMesh(core_axis_name="c", subcore_axis_name="s")
    return pl.kernel(
        _make_sc_body(c0),
        out_type=jax.ShapeDtypeStruct((B, CPC, K), jnp.float32),
        mesh=mesh,
        compiler_params=pltpu.CompilerParams(needs_layout_passes=False),
        scratch_types=(
            [pltpu.VMEM((N, N), jnp.float32)] * 4
            + [pltpu.VMEM((K,), jnp.float32)] * 4
            + [pltpu.SemaphoreType.DMA] * 8
        ),
        name=f"spd_pack_c{c0}",
    )(x)


BB = 1024
KB = 512


def _tc_t_first_body(chunk_ref, out_ref):
    for cc in range(CPC):
        out_ref[cc] = jnp.transpose(chunk_ref[:, cc, :])


def _tc_t_body(buf_ref, chunk_ref, out_ref):
    del buf_ref
    for cc in range(CPC):
        out_ref[cc] = jnp.transpose(chunk_ref[:, cc, :])


def _tc_transpose(chunk, c0, buf):
    grid = ((K + KB - 1) // KB, B // BB)
    in_spec = pl.BlockSpec((BB, CPC, KB), lambda k, b: (b, 0, k))
    out_spec = pl.BlockSpec((CPC, KB, BB), lambda k, b: (c0 // CPC, k, b))
    if buf is None:
        return pl.pallas_call(
            _tc_t_first_body,
            grid=grid,
            in_specs=[in_spec],
            out_specs=out_spec,
            out_shape=jax.ShapeDtypeStruct((C, K, B), jnp.float32),
        )(chunk)
    return pl.pallas_call(
        _tc_t_body,
        grid=grid,
        in_specs=[pl.BlockSpec(memory_space=pltpu.MemorySpace.HBM), in_spec],
        out_specs=out_spec,
        out_shape=jax.ShapeDtypeStruct((C, K, B), jnp.float32),
        input_output_aliases={0: 0},
    )(buf, chunk)


@jax.jit
def _spd_vectorize(x):
    buf = None
    for i in range(NCHUNKS):
        chunk = _sc_chunk(x, i * CPC)
        buf = _tc_transpose(chunk, i * CPC, buf)
    return jnp.transpose(buf, (2, 0, 1))


def kernel(input):
    return _spd_vectorize(input)
